# baseline (speedup 1.0000x reference)
import functools

import jax
import jax.numpy as jnp
from jax import lax
from jax.experimental import pallas as pl
from jax.experimental.pallas import tpu as pltpu
from jax.experimental.pallas import tpu_sc as plsc

N_NODES = 10000
D = 128
N_EDGES = 320000

NC = 2
NS = 16
NW = NC * NS
CH = 32
NCH = 320
NH = 4
HCH = NCH // NH
EPH = (NCH // NH) * CH
EPW = NCH * CH
NPAD = 10112
RPT = NPAD // NS
ZR = 8
NBUF = 8

_sc_mesh = plsc.VectorSubcoreMesh(core_axis_name="c", subcore_axis_name="s")


@functools.partial(
    pl.kernel,
    mesh=_sc_mesh,
    out_type=jax.ShapeDtypeStruct((NC, NPAD, D), jnp.float32),
    scratch_types=[
        pltpu.VMEM((EPH,), jnp.int32),
        pltpu.VMEM((HCH, CH), jnp.int32),
        *[pltpu.VMEM((CH, D), jnp.float32) for _ in range(NBUF)],
        pltpu.VMEM((ZR, D), jnp.float32),
        pltpu.VMEM_SHARED((NPAD, D), jnp.float32),
        *[pltpu.SemaphoreType.DMA for _ in range(NBUF)],
    ],
)
def _sc_spmm(h_hbm, src_hbm, dst_hbm, out_hbm, src_v, dst_v, *rest):
    rows = rest[:NBUF]
    zbuf = rest[NBUF]
    acc = rest[NBUF + 1]
    sems = rest[NBUF + 2:]
    cid = lax.axis_index("c")
    sid = lax.axis_index("s")
    gwid = sid * NC + cid

    for r in range(ZR):
        for l in range(D // 16):
            zbuf[r, pl.ds(l * 16, 16)] = jnp.zeros((16,), jnp.float32)
    row0 = sid * RPT
    for j in range(RPT // ZR):
        pltpu.sync_copy(zbuf, acc.at[pl.ds(row0 + j * ZR, ZR)])
    plsc.subcore_barrier()

    for half in range(NH):
        pltpu.sync_copy(src_hbm.at[gwid, pl.ds(half * EPH, EPH)], src_v)
        pltpu.sync_copy(dst_hbm.at[gwid, pl.ds(half * HCH, HCH)], dst_v)

        for s in range(NBUF):
            pltpu.async_copy(h_hbm.at[src_v.at[pl.ds(s * CH, CH)]],
                             rows[s], sems[s])

        def body(g, carry):
            for s in range(NBUF):
                c = NBUF * g + s
                pltpu.make_async_copy(h_hbm.at[pl.ds(0, CH)], rows[s],
                                      sems[s]).wait()
                pltpu.sync_copy(rows[s], acc.at[dst_v.at[c]], add=True)
                pltpu.async_copy(
                    h_hbm.at[src_v.at[pl.ds((c + NBUF) * CH, CH)]],
                    rows[s], sems[s])
            return carry

        lax.fori_loop(0, HCH // NBUF - 1, body, 0)

        for s in range(NBUF):
            c = HCH - NBUF + s
            pltpu.make_async_copy(h_hbm.at[pl.ds(0, CH)], rows[s],
                                  sems[s]).wait()
            pltpu.sync_copy(rows[s], acc.at[dst_v.at[c]], add=True)

    plsc.subcore_barrier()

    pltpu.sync_copy(acc.at[pl.ds(row0, RPT)],
                    out_hbm.at[cid, pl.ds(row0, RPT)])


BN = 1000


def _tc1_body(pa_ref, pb_ref, h_ref, w0_ref, w1_ref, h1_ref, acc_ref):
    h1 = pa_ref[0] + pb_ref[0]
    h1_ref[...] = h1
    acc_ref[...] = (
        jnp.dot(h_ref[...], w0_ref[...], precision=lax.Precision.HIGHEST,
                preferred_element_type=jnp.float32)
        + jnp.dot(h1, w1_ref[...], precision=lax.Precision.HIGHEST,
                  preferred_element_type=jnp.float32))


def _tc2_body(pa_ref, pb_ref, acc_ref, w2_ref, b_ref, out_ref, *, relu):
    h2 = pa_ref[0] + pb_ref[0]
    o = acc_ref[...] + jnp.dot(h2, w2_ref[...],
                               precision=lax.Precision.HIGHEST,
                               preferred_element_type=jnp.float32)
    o = o + b_ref[...]
    out_ref[...] = jnp.maximum(o, 0.0) if relu else o


_G = N_NODES // BN
_p_spec_a = pl.BlockSpec((1, BN, D), lambda i: (0, i, 0))
_p_spec_b = pl.BlockSpec((1, BN, D), lambda i: (1, i, 0))
_row_spec = pl.BlockSpec((BN, D), lambda i: (i, 0))
_w_spec = pl.BlockSpec((D, D), lambda i: (0, 0))
_b_spec = pl.BlockSpec((1, D), lambda i: (0, 0))


def _tc1(p, h, w0, w1):
    return pl.pallas_call(
        _tc1_body,
        grid=(_G,),
        in_specs=[_p_spec_a, _p_spec_b, _row_spec, _w_spec, _w_spec],
        out_specs=[_row_spec, _row_spec],
        out_shape=[jax.ShapeDtypeStruct((N_NODES, D), jnp.float32),
                   jax.ShapeDtypeStruct((N_NODES, D), jnp.float32)],
    )(p, p, h, w0, w1)


def _tc2(p, acc, w2, b, relu):
    return pl.pallas_call(
        functools.partial(_tc2_body, relu=relu),
        grid=(_G,),
        in_specs=[_p_spec_a, _p_spec_b, _row_spec, _w_spec, _b_spec],
        out_specs=_row_spec,
        out_shape=jax.ShapeDtypeStruct((N_NODES, D), jnp.float32),
    )(p, p, acc, w2, b.reshape(1, D))


def kernel(x, edge_index, Ws, bs):
    ei = edge_index.astype(jnp.int32)
    npad_e = NW * EPW - N_EDGES
    ppw = npad_e // NW
    rpw = N_EDGES // NW
    src_pad = jnp.zeros((NW, ppw), jnp.int32)
    dst_pad = jnp.broadcast_to(
        (jnp.arange(ppw, dtype=jnp.int32) % (NPAD - N_NODES)) + N_NODES,
        (NW, ppw))
    src2 = jnp.concatenate([ei[0].reshape(NW, rpw), src_pad], axis=1)
    dst3 = jnp.concatenate([ei[1].reshape(NW, rpw), dst_pad],
                           axis=1).reshape(NW, NCH, CH)
    h = x
    num_layers = Ws.shape[0]
    for layer in range(num_layers):
        p = _sc_spmm(h, src2, dst3)
        h1, acc = _tc1(p, h, Ws[layer, 0], Ws[layer, 1])
        p2 = _sc_spmm(h1, src2, dst3)
        h = _tc2(p2, acc, Ws[layer, 2], bs[layer], layer < num_layers - 1)
    return h


# speedup vs baseline: 1.0140x; 1.0140x over previous
import functools

import jax
import jax.numpy as jnp
from jax import lax
from jax.experimental import pallas as pl
from jax.experimental.pallas import tpu as pltpu
from jax.experimental.pallas import tpu_sc as plsc

N_NODES = 10000
D = 128
N_EDGES = 320000

NC = 2
NS = 16
NW = NC * NS
CH = 64
NCH = 160
NH = 2
HCH = NCH // NH
EPH = (NCH // NH) * CH
EPW = NCH * CH
NPAD = 10112
RPT = NPAD // NS
ZR = 8
NBUF = 4

_sc_mesh = plsc.VectorSubcoreMesh(core_axis_name="c", subcore_axis_name="s")


@functools.partial(
    pl.kernel,
    mesh=_sc_mesh,
    out_type=jax.ShapeDtypeStruct((NC, NPAD, D), jnp.float32),
    scratch_types=[
        pltpu.VMEM((EPH,), jnp.int32),
        pltpu.VMEM((HCH, CH), jnp.int32),
        pltpu.VMEM((CH, D), jnp.float32),
        pltpu.VMEM((CH, D), jnp.float32),
        pltpu.VMEM((CH, D), jnp.float32),
        pltpu.VMEM((CH, D), jnp.float32),
        pltpu.VMEM((ZR, D), jnp.float32),
        pltpu.VMEM_SHARED((NPAD, D), jnp.float32),
        pltpu.SemaphoreType.DMA,
        pltpu.SemaphoreType.DMA,
        pltpu.SemaphoreType.DMA,
        pltpu.SemaphoreType.DMA,
    ],
)
def _sc_spmm(h_hbm, src_hbm, dst_hbm, out_hbm, src_v, dst_v, rows0, rows1,
             rows2, rows3, zbuf, acc, sem0, sem1, sem2, sem3):
    cid = lax.axis_index("c")
    sid = lax.axis_index("s")
    gwid = sid * NC + cid

    for r in range(ZR):
        for l in range(D // 16):
            zbuf[r, pl.ds(l * 16, 16)] = jnp.zeros((16,), jnp.float32)
    row0 = sid * RPT
    for j in range(RPT // ZR):
        pltpu.sync_copy(zbuf, acc.at[pl.ds(row0 + j * ZR, ZR)])
    plsc.subcore_barrier()

    rows = (rows0, rows1, rows2, rows3)
    sems = (sem0, sem1, sem2, sem3)

    for half in range(NH):
        pltpu.sync_copy(src_hbm.at[gwid, pl.ds(half * EPH, EPH)], src_v)
        pltpu.sync_copy(dst_hbm.at[gwid, pl.ds(half * HCH, HCH)], dst_v)

        for s in range(NBUF):
            pltpu.async_copy(h_hbm.at[src_v.at[pl.ds(s * CH, CH)]],
                             rows[s], sems[s])

        def body(g, carry):
            for s in range(NBUF):
                c = NBUF * g + s
                pltpu.make_async_copy(h_hbm.at[pl.ds(0, CH)], rows[s],
                                      sems[s]).wait()
                pltpu.sync_copy(rows[s], acc.at[dst_v.at[c]], add=True)
                pltpu.async_copy(
                    h_hbm.at[src_v.at[pl.ds((c + NBUF) * CH, CH)]],
                    rows[s], sems[s])
            return carry

        lax.fori_loop(0, HCH // NBUF - 1, body, 0)

        for s in range(NBUF):
            c = HCH - NBUF + s
            pltpu.make_async_copy(h_hbm.at[pl.ds(0, CH)], rows[s],
                                  sems[s]).wait()
            pltpu.sync_copy(rows[s], acc.at[dst_v.at[c]], add=True)

    plsc.subcore_barrier()

    pltpu.sync_copy(acc.at[pl.ds(row0, RPT)],
                    out_hbm.at[cid, pl.ds(row0, RPT)])


BN = 1000


def _tc1_body(pa_ref, pb_ref, h_ref, w0_ref, w1_ref, h1_ref, acc_ref):
    h1 = pa_ref[0] + pb_ref[0]
    h1_ref[...] = h1
    acc_ref[...] = (
        jnp.dot(h_ref[...], w0_ref[...], precision=lax.Precision.HIGHEST,
                preferred_element_type=jnp.float32)
        + jnp.dot(h1, w1_ref[...], precision=lax.Precision.HIGHEST,
                  preferred_element_type=jnp.float32))


def _tc2_body(pa_ref, pb_ref, acc_ref, w2_ref, b_ref, out_ref, *, relu):
    h2 = pa_ref[0] + pb_ref[0]
    o = acc_ref[...] + jnp.dot(h2, w2_ref[...],
                               precision=lax.Precision.HIGHEST,
                               preferred_element_type=jnp.float32)
    o = o + b_ref[...]
    out_ref[...] = jnp.maximum(o, 0.0) if relu else o


_G = N_NODES // BN
_p_spec_a = pl.BlockSpec((1, BN, D), lambda i: (0, i, 0))
_p_spec_b = pl.BlockSpec((1, BN, D), lambda i: (1, i, 0))
_row_spec = pl.BlockSpec((BN, D), lambda i: (i, 0))
_w_spec = pl.BlockSpec((D, D), lambda i: (0, 0))
_b_spec = pl.BlockSpec((1, D), lambda i: (0, 0))


def _tc1(p, h, w0, w1):
    return pl.pallas_call(
        _tc1_body,
        grid=(_G,),
        in_specs=[_p_spec_a, _p_spec_b, _row_spec, _w_spec, _w_spec],
        out_specs=[_row_spec, _row_spec],
        out_shape=[jax.ShapeDtypeStruct((N_NODES, D), jnp.float32),
                   jax.ShapeDtypeStruct((N_NODES, D), jnp.float32)],
    )(p, p, h, w0, w1)


def _tc2(p, acc, w2, b, relu):
    return pl.pallas_call(
        functools.partial(_tc2_body, relu=relu),
        grid=(_G,),
        in_specs=[_p_spec_a, _p_spec_b, _row_spec, _w_spec, _b_spec],
        out_specs=_row_spec,
        out_shape=jax.ShapeDtypeStruct((N_NODES, D), jnp.float32),
    )(p, p, acc, w2, b.reshape(1, D))


def kernel(x, edge_index, Ws, bs):
    ei = edge_index.astype(jnp.int32)
    npad_e = NW * EPW - N_EDGES
    ppw = npad_e // NW
    rpw = N_EDGES // NW
    src_pad = jnp.zeros((NW, ppw), jnp.int32)
    dst_pad = jnp.broadcast_to(
        (jnp.arange(ppw, dtype=jnp.int32) % (NPAD - N_NODES)) + N_NODES,
        (NW, ppw))
    src2 = jnp.concatenate([ei[0].reshape(NW, rpw), src_pad], axis=1)
    dst3 = jnp.concatenate([ei[1].reshape(NW, rpw), dst_pad],
                           axis=1).reshape(NW, NCH, CH)
    h = x
    num_layers = Ws.shape[0]
    for layer in range(num_layers):
        p = _sc_spmm(h, src2, dst3)
        h1, acc = _tc1(p, h, Ws[layer, 0], Ws[layer, 1])
        p2 = _sc_spmm(h1, src2, dst3)
        h = _tc2(p2, acc, Ws[layer, 2], bs[layer], layer < num_layers - 1)
    return h
